# baseline (speedup 1.0000x reference)
import functools

import jax
import jax.numpy as jnp
from jax import lax
from jax.experimental import pallas as pl
from jax.experimental.pallas import tpu as pltpu
from jax.experimental.pallas import tpu_sc as plsc

N = 10000
D = 512
K = 33
KST = 48
NPAD = 10240
R = 128
NB = NPAD // R
E = N * K
NW = 32
RW = NPAD // NW
CHUNK = 10320
EPAD = CHUNK * NW
NC16 = N // 16
NEG = float("-inf")
BIGI = 1 << 30
NSTAT = 8


def _mlp_body(f_ref, w1t_ref, b1_ref, w2t_ref, b2_ref, emb_ref, xn_ref):
    f = f_ref[...]
    h = jnp.dot(f, w1t_ref[...], preferred_element_type=jnp.float32)
    h = jnp.maximum(h + b1_ref[...], 0.0)
    e = jnp.dot(h, w2t_ref[...], preferred_element_type=jnp.float32)
    e = e + b2_ref[...]
    emb_ref[...] = e
    nrm = jnp.sqrt(jnp.sum(e * e, axis=1, keepdims=True))
    xn_ref[...] = e / jnp.maximum(nrm, 1e-12)


def _mlp_call(f, w1t, b1r, w2t, b2r):
    return pl.pallas_call(
        _mlp_body,
        grid=(NB,),
        in_specs=[
            pl.BlockSpec((R, D), lambda i: (i, 0)),
            pl.BlockSpec((D, D), lambda i: (0, 0)),
            pl.BlockSpec((1, D), lambda i: (0, 0)),
            pl.BlockSpec((D, D), lambda i: (0, 0)),
            pl.BlockSpec((1, D), lambda i: (0, 0)),
        ],
        out_specs=[
            pl.BlockSpec((R, D), lambda i: (i, 0)),
            pl.BlockSpec((R, D), lambda i: (i, 0)),
        ],
        out_shape=[
            jax.ShapeDtypeStruct((NPAD, D), jnp.float32),
            jax.ShapeDtypeStruct((NPAD, D), jnp.float32),
        ],
    )(f, w1t, b1r, w2t, b2r)


def _sim_body(xb_ref, xnt_ref, out_ref):
    out_ref[...] = jnp.dot(xb_ref[...], xnt_ref[...],
                           preferred_element_type=jnp.float32)


def _sim_call(xn, xnt):
    return pl.pallas_call(
        _sim_body,
        grid=(NB,),
        in_specs=[
            pl.BlockSpec((R, D), lambda i: (i, 0)),
            pl.BlockSpec((D, NPAD), lambda i: (0, 0)),
        ],
        out_specs=pl.BlockSpec((R, NPAD), lambda i: (i, 0)),
        out_shape=jax.ShapeDtypeStruct((NPAD, NPAD), jnp.float32),
    )(xn, xnt)


def _sctopk_body(sim_hbm, vals_hbm, inds_hbm, pn_hbm,
                 rowbuf, cand_v, candi_v, norm_v, valout, indout):
    wid = lax.axis_index("s") * 2 + lax.axis_index("c")
    base_row = wid * RW
    i16 = lax.broadcasted_iota(jnp.int32, (16,), 0)
    negv = jnp.zeros((16,), jnp.float32) + NEG
    lane0 = i16 == 0

    def zinit(q, c):
        norm_v[pl.ds(q * 16, 16)] = jnp.zeros((16,), jnp.float32)
        return c
    lax.fori_loop(0, (NPAD + 16) // 16, zinit, 0)

    def row_body(r, carry0):
        pltpu.sync_copy(sim_hbm.at[base_row + r], rowbuf)

        def p1(i, bs):
            b0, b1, b2, b3 = bs
            o = i * 64
            return (jnp.maximum(b0, rowbuf[pl.ds(o, 16)]),
                    jnp.maximum(b1, rowbuf[pl.ds(o + 16, 16)]),
                    jnp.maximum(b2, rowbuf[pl.ds(o + 32, 16)]),
                    jnp.maximum(b3, rowbuf[pl.ds(o + 48, 16)]))
        b0, b1, b2, b3 = lax.fori_loop(0, 156, p1, (negv, negv, negv, negv))
        b0 = jnp.maximum(b0, rowbuf[pl.ds(156 * 64, 16)])

        def tsel(j, st):
            c0, c1, c2, c3, _ = st
            mm = jnp.maximum(jnp.maximum(c0, c1), jnp.maximum(c2, c3))
            m = jnp.max(mm)
            return (jnp.where(c0 == m, negv, c0), jnp.where(c1 == m, negv, c1),
                    jnp.where(c2 == m, negv, c2), jnp.where(c3 == m, negv, c3),
                    m)
        t = lax.fori_loop(0, K, tsel, (b0, b1, b2, b3, jnp.float32(0)))[4]

        for q in range(NSTAT + 1):
            cand_v[pl.ds(q * 16, 16)] = negv

        def p2(i, ptr):
            v = rowbuf[pl.ds(i * 16, 16)]

            def dostore(p):
                msk = v >= t
                cnt = jnp.max(plsc.all_reduce_population_count(msk))
                plsc.store_compressed(cand_v.at[pl.ds(p, 16)], v, mask=msk)
                plsc.store_compressed(candi_v.at[pl.ds(p, 16)], i16 + i * 16,
                                      mask=msk)
                return p + cnt
            return lax.cond(jnp.max(v) >= t, dostore, lambda p: p, ptr)
        ptr = lax.fori_loop(0, NC16, p2, jnp.int32(0))
        cand_v[pl.ds(ptr, 16)] = negv
        nv = (ptr + 15) // 16

        ob = r * KST
        valout[pl.ds(ob + 32, 16)] = jnp.zeros((16,), jnp.float32)
        indout[pl.ds(ob + 32, 16)] = i16 + NPAD

        def extract(j, nr):
            mm = negv
            for q in range(NSTAT):
                mm = jnp.maximum(mm, cand_v[pl.ds(q * 16, 16)])

            def dmax(q, acc):
                return jnp.maximum(acc, cand_v[pl.ds(q * 16, 16)])
            mm = lax.fori_loop(NSTAT, nv, dmax, mm)
            m = jnp.max(mm)

            best = jnp.int32(BIGI)
            for q in range(NSTAT):
                v = cand_v[pl.ds(q * 16, 16)]
                iv = candi_v[pl.ds(q * 16, 16)]
                best = jnp.minimum(best, jnp.min(jnp.where(v == m, iv, BIGI)))

            def didx(q, acc):
                v = cand_v[pl.ds(q * 16, 16)]
                iv = candi_v[pl.ds(q * 16, 16)]
                return jnp.minimum(acc,
                                   jnp.min(jnp.where(v == m, iv, BIGI)))
            best = lax.fori_loop(NSTAT, nv, didx, best)

            for q in range(NSTAT):
                v = cand_v[pl.ds(q * 16, 16)]
                iv = candi_v[pl.ds(q * 16, 16)]
                cand_v[pl.ds(q * 16, 16)] = jnp.where(iv == best, negv, v)

            def dmask(q, c):
                v = cand_v[pl.ds(q * 16, 16)]
                iv = candi_v[pl.ds(q * 16, 16)]
                cand_v[pl.ds(q * 16, 16)] = jnp.where(iv == best, negv, v)
                return c
            lax.fori_loop(NSTAT, nv, dmask, 0)

            mv = jnp.zeros((16,), jnp.float32) + m
            bv = jnp.zeros((16,), jnp.int32) + best
            plsc.store_compressed(valout.at[pl.ds(ob + j, 16)], mv, mask=lane0)
            plsc.store_compressed(indout.at[pl.ds(ob + j, 16)], bv, mask=lane0)
            return nr + m

        nr = lax.fori_loop(0, K, extract, jnp.float32(0))

        @pl.when(base_row + r < N)
        def _acc():
            for q in range(3):
                v = valout[pl.ds(ob + q * 16, 16)]
                iv = indout[pl.ds(ob + q * 16, 16)]
                plsc.addupdate_scatter(norm_v, [iv], v)
            plsc.addupdate_scatter(norm_v, [i16 + (base_row + r)],
                                   jnp.zeros((16,), jnp.float32) + nr,
                                   mask=lane0)
        return carry0

    lax.fori_loop(0, RW, row_body, 0)

    pltpu.sync_copy(valout, vals_hbm.at[pl.ds(base_row * KST, RW * KST)])
    pltpu.sync_copy(indout, inds_hbm.at[pl.ds(base_row * KST, RW * KST)])
    pltpu.sync_copy(norm_v.at[pl.ds(0, NPAD)], pn_hbm.at[wid])


def _sctopk_call(sim):
    call = functools.partial(
        pl.kernel,
        mesh=plsc.VectorSubcoreMesh(core_axis_name="c", subcore_axis_name="s"),
        compiler_params=pltpu.CompilerParams(needs_layout_passes=False),
        out_type=[
            jax.ShapeDtypeStruct((NPAD * KST,), jnp.float32),
            jax.ShapeDtypeStruct((NPAD * KST,), jnp.int32),
            jax.ShapeDtypeStruct((NW, NPAD), jnp.float32),
        ],
        scratch_types=[
            pltpu.VMEM((NPAD,), jnp.float32),
            pltpu.VMEM((NPAD,), jnp.float32),
            pltpu.VMEM((NPAD,), jnp.int32),
            pltpu.VMEM((NPAD + 16,), jnp.float32),
            pltpu.VMEM((RW * KST,), jnp.float32),
            pltpu.VMEM((RW * KST,), jnp.int32),
        ],
    )(_sctopk_body)
    return call(sim)


def _norm_body(pn_ref, rn_ref):
    rn_ref[...] = lax.rsqrt(jnp.sum(pn_ref[...], axis=0, keepdims=True))


def _norm_call(pn):
    return pl.pallas_call(
        _norm_body,
        out_shape=jax.ShapeDtypeStruct((1, NPAD), jnp.float32),
    )(pn)


def _edge_body(vals_hbm, rows_hbm, cols_hbm, rn_hbm, out_hbm,
               vals_v, rows_v, cols_v, rn_v, out_v):
    wid = lax.axis_index("s") * 2 + lax.axis_index("c")
    base = wid * CHUNK
    pltpu.sync_copy(rn_hbm, rn_v)
    pltpu.sync_copy(vals_hbm.at[pl.ds(base, CHUNK)], vals_v)
    pltpu.sync_copy(rows_hbm.at[pl.ds(base, CHUNK)], rows_v)
    pltpu.sync_copy(cols_hbm.at[pl.ds(base, CHUNK)], cols_v)

    def body(i, carry):
        s = pl.ds(i * 16, 16)
        c = cols_v[s]
        r = rows_v[s]
        v = vals_v[s]
        rc = plsc.load_gather(rn_v, [c])
        rr = plsc.load_gather(rn_v, [r])
        out_v[s] = v * rc * rr
        return carry

    lax.fori_loop(0, CHUNK // 16, body, 0)
    pltpu.sync_copy(out_v, out_hbm.at[pl.ds(base, CHUNK)])


def _edge_call(*args):
    call = functools.partial(
        pl.kernel,
        mesh=plsc.VectorSubcoreMesh(core_axis_name="c", subcore_axis_name="s"),
        compiler_params=pltpu.CompilerParams(needs_layout_passes=False),
        out_type=jax.ShapeDtypeStruct((EPAD,), jnp.float32),
        scratch_types=[
            pltpu.VMEM((CHUNK,), jnp.float32),
            pltpu.VMEM((CHUNK,), jnp.int32),
            pltpu.VMEM((CHUNK,), jnp.int32),
            pltpu.VMEM((NPAD,), jnp.float32),
            pltpu.VMEM((CHUNK,), jnp.float32),
        ],
    )(_edge_body)
    return call(*args)


def kernel(features, W1, b1, W2, b2):
    f = jnp.pad(features, ((0, NPAD - N), (0, 0)))
    emb, xn = _mlp_call(f, W1.T, b1.reshape(1, D), W2.T, b2.reshape(1, D))
    sim = _sim_call(xn, xn.T)
    vals48, inds48, pn = _sctopk_call(sim)
    rn = _norm_call(pn)

    vals = vals48.reshape(NPAD, KST)[:N, :K]
    inds = inds48.reshape(NPAD, KST)[:N, :K]
    rows = jnp.repeat(jnp.arange(N, dtype=jnp.int32), K)
    cols = inds.reshape(-1)
    vflat = vals.reshape(-1)

    w = _edge_call(
        jnp.pad(vflat, (0, EPAD - E)),
        jnp.pad(rows, (0, EPAD - E)),
        jnp.pad(cols, (0, EPAD - E)),
        rn.reshape(NPAD),
    )
    edge_weight = w[:E]
    edge_index = jnp.stack([rows, cols])
    return (edge_index, edge_weight, emb[:N])


# speedup vs baseline: 1.1639x; 1.1639x over previous
import functools

import jax
import jax.numpy as jnp
from jax import lax
from jax.experimental import pallas as pl
from jax.experimental.pallas import tpu as pltpu
from jax.experimental.pallas import tpu_sc as plsc

N = 10000
D = 512
K = 33
KST = 48
NPAD = 10240
R = 128
NB = NPAD // R
E = N * K
NW = 32
RW = NPAD // NW
CHUNK = 10320
EPAD = CHUNK * NW
NC16 = N // 16
NEG = float("-inf")
BIGI = 1 << 30
NSTAT = 8


def _mlp_body(f_ref, w1t_ref, b1_ref, w2t_ref, b2_ref, emb_ref, xn_ref):
    f = f_ref[...]
    h = jnp.dot(f, w1t_ref[...], preferred_element_type=jnp.float32)
    h = jnp.maximum(h + b1_ref[...], 0.0)
    e = jnp.dot(h, w2t_ref[...], preferred_element_type=jnp.float32)
    e = e + b2_ref[...]
    emb_ref[...] = e
    nrm = jnp.sqrt(jnp.sum(e * e, axis=1, keepdims=True))
    xn_ref[...] = e / jnp.maximum(nrm, 1e-12)


def _mlp_call(f, w1t, b1r, w2t, b2r):
    return pl.pallas_call(
        _mlp_body,
        grid=(NB,),
        in_specs=[
            pl.BlockSpec((R, D), lambda i: (i, 0)),
            pl.BlockSpec((D, D), lambda i: (0, 0)),
            pl.BlockSpec((1, D), lambda i: (0, 0)),
            pl.BlockSpec((D, D), lambda i: (0, 0)),
            pl.BlockSpec((1, D), lambda i: (0, 0)),
        ],
        out_specs=[
            pl.BlockSpec((R, D), lambda i: (i, 0)),
            pl.BlockSpec((R, D), lambda i: (i, 0)),
        ],
        out_shape=[
            jax.ShapeDtypeStruct((NPAD, D), jnp.float32),
            jax.ShapeDtypeStruct((NPAD, D), jnp.float32),
        ],
    )(f, w1t, b1r, w2t, b2r)


def _sim_body(xb_ref, xnt_ref, out_ref):
    out_ref[...] = jnp.dot(xb_ref[...], xnt_ref[...],
                           preferred_element_type=jnp.float32)


def _sim_call(xn, xnt):
    return pl.pallas_call(
        _sim_body,
        grid=(NB,),
        in_specs=[
            pl.BlockSpec((R, D), lambda i: (i, 0)),
            pl.BlockSpec((D, NPAD), lambda i: (0, 0)),
        ],
        out_specs=pl.BlockSpec((R, NPAD), lambda i: (i, 0)),
        out_shape=jax.ShapeDtypeStruct((NPAD, NPAD), jnp.float32),
    )(xn, xnt)


def _sctopk_body(sim_hbm, vals_hbm, inds_hbm, pn_hbm,
                 rowbuf, cand_v, candi_v, norm_v, valout, indout):
    wid = lax.axis_index("s") * 2 + lax.axis_index("c")
    base_row = wid * RW
    i16 = lax.broadcasted_iota(jnp.int32, (16,), 0)
    negv = jnp.zeros((16,), jnp.float32) + NEG
    lane0 = i16 == 0

    def zinit(q, c):
        norm_v[pl.ds(q * 16, 16)] = jnp.zeros((16,), jnp.float32)
        return c
    lax.fori_loop(0, (NPAD + 16) // 16, zinit, 0)

    def row_body(r, carry0):
        pltpu.sync_copy(sim_hbm.at[base_row + r], rowbuf)
        for q in range(15):
            rowbuf[pl.ds(N + q * 16, 16)] = negv

        def p1(i, bs):
            b0, b1, b2, b3 = bs
            o = i * 64
            return (jnp.maximum(b0, rowbuf[pl.ds(o, 16)]),
                    jnp.maximum(b1, rowbuf[pl.ds(o + 16, 16)]),
                    jnp.maximum(b2, rowbuf[pl.ds(o + 32, 16)]),
                    jnp.maximum(b3, rowbuf[pl.ds(o + 48, 16)]))
        b0, b1, b2, b3 = lax.fori_loop(0, 160, p1, (negv, negv, negv, negv))

        def tsel(j, st):
            c0, c1, c2, c3, _ = st
            mm = jnp.maximum(jnp.maximum(c0, c1), jnp.maximum(c2, c3))
            m = jnp.max(mm)
            return (jnp.where(c0 == m, negv, c0), jnp.where(c1 == m, negv, c1),
                    jnp.where(c2 == m, negv, c2), jnp.where(c3 == m, negv, c3),
                    m)
        t = lax.fori_loop(0, K, tsel, (b0, b1, b2, b3, jnp.float32(0)))[4]

        for q in range(NSTAT + 1):
            cand_v[pl.ds(q * 16, 16)] = negv

        def p2(g, ptr):
            o = g * 64
            v0 = rowbuf[pl.ds(o, 16)]
            v1 = rowbuf[pl.ds(o + 16, 16)]
            v2 = rowbuf[pl.ds(o + 32, 16)]
            v3 = rowbuf[pl.ds(o + 48, 16)]
            gm = jnp.maximum(jnp.maximum(v0, v1), jnp.maximum(v2, v3))

            def dostore(p):
                for off in (0, 16, 32, 48):
                    vv = rowbuf[pl.ds(o + off, 16)]
                    msk = vv >= t
                    cnt = jnp.max(plsc.all_reduce_population_count(msk))
                    plsc.store_compressed(cand_v.at[pl.ds(p, 16)], vv,
                                          mask=msk)
                    plsc.store_compressed(candi_v.at[pl.ds(p, 16)],
                                          i16 + (o + off), mask=msk)
                    p = p + cnt
                return p
            return lax.cond(jnp.max(gm) >= t, dostore, lambda p: p, ptr)
        ptr = lax.fori_loop(0, 160, p2, jnp.int32(0))
        cand_v[pl.ds(ptr, 16)] = negv
        nv = (ptr + 15) // 16

        ob = r * KST
        valout[pl.ds(ob + 32, 16)] = jnp.zeros((16,), jnp.float32)
        indout[pl.ds(ob + 32, 16)] = i16 + NPAD

        def extract(j, nr):
            mm = negv
            for q in range(NSTAT):
                mm = jnp.maximum(mm, cand_v[pl.ds(q * 16, 16)])

            def dmax(q, acc):
                return jnp.maximum(acc, cand_v[pl.ds(q * 16, 16)])
            mm = lax.fori_loop(NSTAT, nv, dmax, mm)
            m = jnp.max(mm)

            best = jnp.int32(BIGI)
            for q in range(NSTAT):
                v = cand_v[pl.ds(q * 16, 16)]
                iv = candi_v[pl.ds(q * 16, 16)]
                best = jnp.minimum(best, jnp.min(jnp.where(v == m, iv, BIGI)))

            def didx(q, acc):
                v = cand_v[pl.ds(q * 16, 16)]
                iv = candi_v[pl.ds(q * 16, 16)]
                return jnp.minimum(acc,
                                   jnp.min(jnp.where(v == m, iv, BIGI)))
            best = lax.fori_loop(NSTAT, nv, didx, best)

            for q in range(NSTAT):
                v = cand_v[pl.ds(q * 16, 16)]
                iv = candi_v[pl.ds(q * 16, 16)]
                cand_v[pl.ds(q * 16, 16)] = jnp.where(iv == best, negv, v)

            def dmask(q, c):
                v = cand_v[pl.ds(q * 16, 16)]
                iv = candi_v[pl.ds(q * 16, 16)]
                cand_v[pl.ds(q * 16, 16)] = jnp.where(iv == best, negv, v)
                return c
            lax.fori_loop(NSTAT, nv, dmask, 0)

            mv = jnp.zeros((16,), jnp.float32) + m
            bv = jnp.zeros((16,), jnp.int32) + best
            plsc.store_compressed(valout.at[pl.ds(ob + j, 16)], mv, mask=lane0)
            plsc.store_compressed(indout.at[pl.ds(ob + j, 16)], bv, mask=lane0)
            return nr + m

        nr = lax.fori_loop(0, K, extract, jnp.float32(0))

        @pl.when(base_row + r < N)
        def _acc():
            for q in range(3):
                v = valout[pl.ds(ob + q * 16, 16)]
                iv = indout[pl.ds(ob + q * 16, 16)]
                plsc.addupdate_scatter(norm_v, [iv], v)
            plsc.addupdate_scatter(norm_v, [i16 + (base_row + r)],
                                   jnp.zeros((16,), jnp.float32) + nr,
                                   mask=lane0)
        return carry0

    lax.fori_loop(0, RW, row_body, 0)

    pltpu.sync_copy(valout, vals_hbm.at[pl.ds(base_row * KST, RW * KST)])
    pltpu.sync_copy(indout, inds_hbm.at[pl.ds(base_row * KST, RW * KST)])
    pltpu.sync_copy(norm_v.at[pl.ds(0, NPAD)], pn_hbm.at[wid])


def _sctopk_call(sim):
    call = functools.partial(
        pl.kernel,
        mesh=plsc.VectorSubcoreMesh(core_axis_name="c", subcore_axis_name="s"),
        compiler_params=pltpu.CompilerParams(needs_layout_passes=False),
        out_type=[
            jax.ShapeDtypeStruct((NPAD * KST,), jnp.float32),
            jax.ShapeDtypeStruct((NPAD * KST,), jnp.int32),
            jax.ShapeDtypeStruct((NW, NPAD), jnp.float32),
        ],
        scratch_types=[
            pltpu.VMEM((NPAD,), jnp.float32),
            pltpu.VMEM((NPAD,), jnp.float32),
            pltpu.VMEM((NPAD,), jnp.int32),
            pltpu.VMEM((NPAD + 16,), jnp.float32),
            pltpu.VMEM((RW * KST,), jnp.float32),
            pltpu.VMEM((RW * KST,), jnp.int32),
        ],
    )(_sctopk_body)
    return call(sim)


def _norm_body(pn_ref, rn_ref):
    rn_ref[...] = lax.rsqrt(jnp.sum(pn_ref[...], axis=0, keepdims=True))


def _norm_call(pn):
    return pl.pallas_call(
        _norm_body,
        out_shape=jax.ShapeDtypeStruct((1, NPAD), jnp.float32),
    )(pn)


def _edge_body(vals_hbm, rows_hbm, cols_hbm, rn_hbm, out_hbm,
               vals_v, rows_v, cols_v, rn_v, out_v):
    wid = lax.axis_index("s") * 2 + lax.axis_index("c")
    base = wid * CHUNK
    pltpu.sync_copy(rn_hbm, rn_v)
    pltpu.sync_copy(vals_hbm.at[pl.ds(base, CHUNK)], vals_v)
    pltpu.sync_copy(rows_hbm.at[pl.ds(base, CHUNK)], rows_v)
    pltpu.sync_copy(cols_hbm.at[pl.ds(base, CHUNK)], cols_v)

    def body(i, carry):
        s = pl.ds(i * 16, 16)
        c = cols_v[s]
        r = rows_v[s]
        v = vals_v[s]
        rc = plsc.load_gather(rn_v, [c])
        rr = plsc.load_gather(rn_v, [r])
        out_v[s] = v * rc * rr
        return carry

    lax.fori_loop(0, CHUNK // 16, body, 0)
    pltpu.sync_copy(out_v, out_hbm.at[pl.ds(base, CHUNK)])


def _edge_call(*args):
    call = functools.partial(
        pl.kernel,
        mesh=plsc.VectorSubcoreMesh(core_axis_name="c", subcore_axis_name="s"),
        compiler_params=pltpu.CompilerParams(needs_layout_passes=False),
        out_type=jax.ShapeDtypeStruct((EPAD,), jnp.float32),
        scratch_types=[
            pltpu.VMEM((CHUNK,), jnp.float32),
            pltpu.VMEM((CHUNK,), jnp.int32),
            pltpu.VMEM((CHUNK,), jnp.int32),
            pltpu.VMEM((NPAD,), jnp.float32),
            pltpu.VMEM((CHUNK,), jnp.float32),
        ],
    )(_edge_body)
    return call(*args)


def kernel(features, W1, b1, W2, b2):
    f = jnp.pad(features, ((0, NPAD - N), (0, 0)))
    emb, xn = _mlp_call(f, W1.T, b1.reshape(1, D), W2.T, b2.reshape(1, D))
    sim = _sim_call(xn, xn.T)
    vals48, inds48, pn = _sctopk_call(sim)
    rn = _norm_call(pn)

    vals = vals48.reshape(NPAD, KST)[:N, :K]
    inds = inds48.reshape(NPAD, KST)[:N, :K]
    rows = jnp.repeat(jnp.arange(N, dtype=jnp.int32), K)
    cols = inds.reshape(-1)
    vflat = vals.reshape(-1)

    w = _edge_call(
        jnp.pad(vflat, (0, EPAD - E)),
        jnp.pad(rows, (0, EPAD - E)),
        jnp.pad(cols, (0, EPAD - E)),
        rn.reshape(NPAD),
    )
    edge_weight = w[:E]
    edge_index = jnp.stack([rows, cols])
    return (edge_index, edge_weight, emb[:N])


# speedup vs baseline: 1.1795x; 1.0133x over previous
import functools

import jax
import jax.numpy as jnp
from jax import lax
from jax.experimental import pallas as pl
from jax.experimental.pallas import tpu as pltpu
from jax.experimental.pallas import tpu_sc as plsc

N = 10000
D = 512
K = 33
KST = 48
NPAD = 10240
R = 128
NB = NPAD // R
E = N * K
NW = 32
RW = NPAD // NW
CHUNK = 10320
EPAD = CHUNK * NW
NC16 = N // 16
NEG = float("-inf")
BIGI = 1 << 30
NSTAT = 8


def _mlp_body(f_ref, w1t_ref, b1_ref, w2t_ref, b2_ref, emb_ref, xn_ref):
    f = f_ref[...]
    h = jnp.dot(f, w1t_ref[...], preferred_element_type=jnp.float32)
    h = jnp.maximum(h + b1_ref[...], 0.0)
    e = jnp.dot(h, w2t_ref[...], preferred_element_type=jnp.float32)
    e = e + b2_ref[...]
    emb_ref[...] = e
    nrm = jnp.sqrt(jnp.sum(e * e, axis=1, keepdims=True))
    xn_ref[...] = e / jnp.maximum(nrm, 1e-12)


def _mlp_call(f, w1t, b1r, w2t, b2r):
    return pl.pallas_call(
        _mlp_body,
        grid=(NB,),
        in_specs=[
            pl.BlockSpec((R, D), lambda i: (i, 0)),
            pl.BlockSpec((D, D), lambda i: (0, 0)),
            pl.BlockSpec((1, D), lambda i: (0, 0)),
            pl.BlockSpec((D, D), lambda i: (0, 0)),
            pl.BlockSpec((1, D), lambda i: (0, 0)),
        ],
        out_specs=[
            pl.BlockSpec((R, D), lambda i: (i, 0)),
            pl.BlockSpec((R, D), lambda i: (i, 0)),
        ],
        out_shape=[
            jax.ShapeDtypeStruct((NPAD, D), jnp.float32),
            jax.ShapeDtypeStruct((NPAD, D), jnp.float32),
        ],
    )(f, w1t, b1r, w2t, b2r)


def _sim_body(xb_ref, xnt_ref, out_ref):
    out_ref[...] = jnp.dot(xb_ref[...], xnt_ref[...],
                           preferred_element_type=jnp.float32)


def _sim_call(xn, xnt):
    return pl.pallas_call(
        _sim_body,
        grid=(NB,),
        in_specs=[
            pl.BlockSpec((R, D), lambda i: (i, 0)),
            pl.BlockSpec((D, NPAD), lambda i: (0, 0)),
        ],
        out_specs=pl.BlockSpec((R, NPAD), lambda i: (i, 0)),
        out_shape=jax.ShapeDtypeStruct((NPAD, NPAD), jnp.float32),
    )(xn, xnt)


def _sctopk_body(sim_hbm, vals_hbm, inds_hbm, pn_hbm,
                 rowbuf, cand_v, candi_v, norm_v, valout, indout):
    wid = lax.axis_index("s") * 2 + lax.axis_index("c")
    base_row = wid * RW
    i16 = lax.broadcasted_iota(jnp.int32, (16,), 0)
    negv = jnp.zeros((16,), jnp.float32) + NEG
    lane0 = i16 == 0

    def zinit(q, c):
        norm_v[pl.ds(q * 16, 16)] = jnp.zeros((16,), jnp.float32)
        return c
    lax.fori_loop(0, (NPAD + 16) // 16, zinit, 0)

    def row_body(r, carry0):
        pltpu.sync_copy(sim_hbm.at[base_row + r], rowbuf)
        for q in range(15):
            rowbuf[pl.ds(N + q * 16, 16)] = negv

        def p1(i, bs):
            b0, b1, b2, b3 = bs
            o = i * 64
            return (jnp.maximum(b0, rowbuf[pl.ds(o, 16)]),
                    jnp.maximum(b1, rowbuf[pl.ds(o + 16, 16)]),
                    jnp.maximum(b2, rowbuf[pl.ds(o + 32, 16)]),
                    jnp.maximum(b3, rowbuf[pl.ds(o + 48, 16)]))
        b0, b1, b2, b3 = lax.fori_loop(0, 160, p1, (negv, negv, negv, negv))

        def tsel(j, st):
            c0, c1, c2, c3, _ = st
            mm = jnp.maximum(jnp.maximum(c0, c1), jnp.maximum(c2, c3))
            m = jnp.max(mm)
            return (jnp.where(c0 == m, negv, c0), jnp.where(c1 == m, negv, c1),
                    jnp.where(c2 == m, negv, c2), jnp.where(c3 == m, negv, c3),
                    m)
        t = lax.fori_loop(0, K, tsel, (b0, b1, b2, b3, jnp.float32(0)))[4]

        for q in range(NSTAT + 1):
            cand_v[pl.ds(q * 16, 16)] = negv

        def p2(g, ptr):
            o = g * 64
            v0 = rowbuf[pl.ds(o, 16)]
            v1 = rowbuf[pl.ds(o + 16, 16)]
            v2 = rowbuf[pl.ds(o + 32, 16)]
            v3 = rowbuf[pl.ds(o + 48, 16)]
            gm = jnp.maximum(jnp.maximum(v0, v1), jnp.maximum(v2, v3))

            def dostore(p):
                for off in (0, 16, 32, 48):
                    vv = rowbuf[pl.ds(o + off, 16)]
                    msk = vv >= t
                    cnt = jnp.max(plsc.all_reduce_population_count(msk))
                    plsc.store_compressed(cand_v.at[pl.ds(p, 16)], vv,
                                          mask=msk)
                    plsc.store_compressed(candi_v.at[pl.ds(p, 16)],
                                          i16 + (o + off), mask=msk)
                    p = p + cnt
                return p
            return lax.cond(jnp.max(gm) >= t, dostore, lambda p: p, ptr)
        ptr = lax.fori_loop(0, 160, p2, jnp.int32(0))
        cand_v[pl.ds(ptr, 16)] = negv
        nv = (ptr + 15) // 16

        ob = r * KST
        valout[pl.ds(ob + 32, 16)] = jnp.zeros((16,), jnp.float32)
        indout[pl.ds(ob + 32, 16)] = i16 + NPAD

        def make_extract(dyn):
            def extract(j, nr):
                mm = negv
                for q in range(NSTAT):
                    mm = jnp.maximum(mm, cand_v[pl.ds(q * 16, 16)])
                if dyn:
                    def dmax(q, acc):
                        return jnp.maximum(acc, cand_v[pl.ds(q * 16, 16)])
                    mm = lax.fori_loop(NSTAT, nv, dmax, mm)
                m = jnp.max(mm)

                best = jnp.int32(BIGI)
                for q in range(NSTAT):
                    v = cand_v[pl.ds(q * 16, 16)]
                    iv = candi_v[pl.ds(q * 16, 16)]
                    best = jnp.minimum(best,
                                       jnp.min(jnp.where(v == m, iv, BIGI)))
                if dyn:
                    def didx(q, acc):
                        v = cand_v[pl.ds(q * 16, 16)]
                        iv = candi_v[pl.ds(q * 16, 16)]
                        return jnp.minimum(
                            acc, jnp.min(jnp.where(v == m, iv, BIGI)))
                    best = lax.fori_loop(NSTAT, nv, didx, best)

                for q in range(NSTAT):
                    v = cand_v[pl.ds(q * 16, 16)]
                    iv = candi_v[pl.ds(q * 16, 16)]
                    cand_v[pl.ds(q * 16, 16)] = jnp.where(iv == best, negv, v)
                if dyn:
                    def dmask(q, c):
                        v = cand_v[pl.ds(q * 16, 16)]
                        iv = candi_v[pl.ds(q * 16, 16)]
                        cand_v[pl.ds(q * 16, 16)] = jnp.where(iv == best,
                                                              negv, v)
                        return c
                    lax.fori_loop(NSTAT, nv, dmask, 0)

                mv = jnp.zeros((16,), jnp.float32) + m
                bv = jnp.zeros((16,), jnp.int32) + best
                plsc.store_compressed(valout.at[pl.ds(ob + j, 16)], mv,
                                      mask=lane0)
                plsc.store_compressed(indout.at[pl.ds(ob + j, 16)], bv,
                                      mask=lane0)
                return nr + m
            return extract

        nr = lax.cond(
            nv <= NSTAT,
            lambda: lax.fori_loop(0, K, make_extract(False), jnp.float32(0)),
            lambda: lax.fori_loop(0, K, make_extract(True), jnp.float32(0)))

        @pl.when(base_row + r < N)
        def _acc():
            for q in range(3):
                v = valout[pl.ds(ob + q * 16, 16)]
                iv = indout[pl.ds(ob + q * 16, 16)]
                plsc.addupdate_scatter(norm_v, [iv], v)
            plsc.addupdate_scatter(norm_v, [i16 + (base_row + r)],
                                   jnp.zeros((16,), jnp.float32) + nr,
                                   mask=lane0)
        return carry0

    lax.fori_loop(0, RW, row_body, 0)

    pltpu.sync_copy(valout, vals_hbm.at[pl.ds(base_row * KST, RW * KST)])
    pltpu.sync_copy(indout, inds_hbm.at[pl.ds(base_row * KST, RW * KST)])
    pltpu.sync_copy(norm_v.at[pl.ds(0, NPAD)], pn_hbm.at[wid])


def _sctopk_call(sim):
    call = functools.partial(
        pl.kernel,
        mesh=plsc.VectorSubcoreMesh(core_axis_name="c", subcore_axis_name="s"),
        compiler_params=pltpu.CompilerParams(needs_layout_passes=False),
        out_type=[
            jax.ShapeDtypeStruct((NPAD * KST,), jnp.float32),
            jax.ShapeDtypeStruct((NPAD * KST,), jnp.int32),
            jax.ShapeDtypeStruct((NW, NPAD), jnp.float32),
        ],
        scratch_types=[
            pltpu.VMEM((NPAD,), jnp.float32),
            pltpu.VMEM((NPAD,), jnp.float32),
            pltpu.VMEM((NPAD,), jnp.int32),
            pltpu.VMEM((NPAD + 16,), jnp.float32),
            pltpu.VMEM((RW * KST,), jnp.float32),
            pltpu.VMEM((RW * KST,), jnp.int32),
        ],
    )(_sctopk_body)
    return call(sim)


def _norm_body(pn_ref, rn_ref):
    rn_ref[...] = lax.rsqrt(jnp.sum(pn_ref[...], axis=0, keepdims=True))


def _norm_call(pn):
    return pl.pallas_call(
        _norm_body,
        out_shape=jax.ShapeDtypeStruct((1, NPAD), jnp.float32),
    )(pn)


def _edge_body(vals_hbm, rows_hbm, cols_hbm, rn_hbm, out_hbm,
               vals_v, rows_v, cols_v, rn_v, out_v):
    wid = lax.axis_index("s") * 2 + lax.axis_index("c")
    base = wid * CHUNK
    pltpu.sync_copy(rn_hbm, rn_v)
    pltpu.sync_copy(vals_hbm.at[pl.ds(base, CHUNK)], vals_v)
    pltpu.sync_copy(rows_hbm.at[pl.ds(base, CHUNK)], rows_v)
    pltpu.sync_copy(cols_hbm.at[pl.ds(base, CHUNK)], cols_v)

    def body(i, carry):
        s = pl.ds(i * 16, 16)
        c = cols_v[s]
        r = rows_v[s]
        v = vals_v[s]
        rc = plsc.load_gather(rn_v, [c])
        rr = plsc.load_gather(rn_v, [r])
        out_v[s] = v * rc * rr
        return carry

    lax.fori_loop(0, CHUNK // 16, body, 0)
    pltpu.sync_copy(out_v, out_hbm.at[pl.ds(base, CHUNK)])


def _edge_call(*args):
    call = functools.partial(
        pl.kernel,
        mesh=plsc.VectorSubcoreMesh(core_axis_name="c", subcore_axis_name="s"),
        compiler_params=pltpu.CompilerParams(needs_layout_passes=False),
        out_type=jax.ShapeDtypeStruct((EPAD,), jnp.float32),
        scratch_types=[
            pltpu.VMEM((CHUNK,), jnp.float32),
            pltpu.VMEM((CHUNK,), jnp.int32),
            pltpu.VMEM((CHUNK,), jnp.int32),
            pltpu.VMEM((NPAD,), jnp.float32),
            pltpu.VMEM((CHUNK,), jnp.float32),
        ],
    )(_edge_body)
    return call(*args)


def kernel(features, W1, b1, W2, b2):
    f = jnp.pad(features, ((0, NPAD - N), (0, 0)))
    emb, xn = _mlp_call(f, W1.T, b1.reshape(1, D), W2.T, b2.reshape(1, D))
    sim = _sim_call(xn, xn.T)
    vals48, inds48, pn = _sctopk_call(sim)
    rn = _norm_call(pn)

    vals = vals48.reshape(NPAD, KST)[:N, :K]
    inds = inds48.reshape(NPAD, KST)[:N, :K]
    rows = jnp.repeat(jnp.arange(N, dtype=jnp.int32), K)
    cols = inds.reshape(-1)
    vflat = vals.reshape(-1)

    w = _edge_call(
        jnp.pad(vflat, (0, EPAD - E)),
        jnp.pad(rows, (0, EPAD - E)),
        jnp.pad(cols, (0, EPAD - E)),
        rn.reshape(NPAD),
    )
    edge_weight = w[:E]
    edge_index = jnp.stack([rows, cols])
    return (edge_index, edge_weight, emb[:N])
